# Initial kernel scaffold; baseline (speedup 1.0000x reference)
import jax
import jax.numpy as jnp
from jax.experimental import pallas as pl


def kernel(h, edge_index, W1, b1, W2, b2, eps, gamma, beta):
    raise NotImplementedError("write your pallas kernel here")


# speedup vs baseline: 5.2201x; 5.2201x over previous
import functools

import jax
import jax.numpy as jnp
from jax import lax
from jax.experimental import pallas as pl
from jax.experimental.pallas import tpu as pltpu
from jax.experimental.pallas import tpu_sc as plsc

_N = 10000
_D = 256
_E = 160000
_HALF = _D // 2
_NTILES = 16
_EDGES_PER_TILE = _E // _NTILES
_CHUNK = 200
_NCHUNK = _EDGES_PER_TILE // _CHUNK
_WB_TILES = 10
_WB_ROWS = _N // _WB_TILES


def _make_sc_agg():
    mesh = plsc.VectorSubcoreMesh(core_axis_name="c", subcore_axis_name="s")

    @functools.partial(
        pl.kernel,
        mesh=mesh,
        out_type=jax.ShapeDtypeStruct((2, _N, _HALF), jnp.float32),
        scratch_types=[
            pltpu.VMEM((_CHUNK,), jnp.int32),
            pltpu.VMEM((_CHUNK,), jnp.int32),
            pltpu.VMEM((_CHUNK, _HALF), jnp.float32),
            pltpu.VMEM_SHARED((_N, _HALF), jnp.float32),
            pltpu.SemaphoreType.DMA,
        ],
    )
    def sc_agg(h2_hbm, idx2_hbm, dst_hbm, zeros_hbm, out_hbm,
               idx_v, dst_v, rows_v, acc_sh, sem):
        c = lax.axis_index("c")
        s = lax.axis_index("s")

        row0 = s * _WB_ROWS

        @pl.when(s < _WB_TILES)
        def _zero():
            pltpu.sync_copy(zeros_hbm, acc_sh.at[pl.ds(row0, _WB_ROWS)])

        plsc.subcore_barrier()

        def chunk_body(k, carry):
            base = s * _EDGES_PER_TILE + k * _CHUNK
            pltpu.sync_copy(idx2_hbm.at[pl.ds(c * _E + base, _CHUNK)], idx_v)
            pltpu.sync_copy(dst_hbm.at[pl.ds(base, _CHUNK)], dst_v)
            pltpu.async_copy(h2_hbm.at[idx_v], rows_v, sem).wait()
            pltpu.sync_copy(rows_v, acc_sh.at[dst_v], add=True)
            return carry

        lax.fori_loop(0, _NCHUNK, chunk_body, 0)
        plsc.subcore_barrier()

        @pl.when(s < _WB_TILES)
        def _writeback():
            pltpu.sync_copy(acc_sh.at[pl.ds(row0, _WB_ROWS)],
                            out_hbm.at[c, pl.ds(row0, _WB_ROWS)])

    return sc_agg


_sc_agg = _make_sc_agg()


def _dense_body(h_ref, a0_ref, a1_ref, w1_ref, b1_ref, w2_ref, b2_ref,
                sc_ref, gamma_ref, beta_ref, out_ref):
    h = h_ref[...]
    agg = jnp.concatenate([a0_ref[...], a1_ref[...]], axis=1)
    z = h * sc_ref[...] + agg
    t = jnp.maximum(jnp.dot(z, w1_ref[...], preferred_element_type=jnp.float32)
                    + b1_ref[...], 0.0)
    r = jnp.dot(t, w2_ref[...], preferred_element_type=jnp.float32) \
        + b2_ref[...] + h
    mu = jnp.mean(r, axis=1, keepdims=True)
    d = r - mu
    var = jnp.mean(d * d, axis=1, keepdims=True)
    ln = d * lax.rsqrt(var + 1e-5) * gamma_ref[...] + beta_ref[...]
    out_ref[...] = jnp.maximum(ln, 0.0)


_BLK = 1000


def _dense(h, a0, a1, W1, b1, W2, b2, scale, gamma, beta):
    nblk = _N // _BLK
    full = lambda i: (0, 0)
    return pl.pallas_call(
        _dense_body,
        grid=(nblk,),
        in_specs=[
            pl.BlockSpec((_BLK, _D), lambda i: (i, 0)),
            pl.BlockSpec((_BLK, _HALF), lambda i: (i, 0)),
            pl.BlockSpec((_BLK, _HALF), lambda i: (i, 0)),
            pl.BlockSpec((_D, _D), full),
            pl.BlockSpec((1, _D), full),
            pl.BlockSpec((_D, _D), full),
            pl.BlockSpec((1, _D), full),
            pl.BlockSpec((1, 1), full),
            pl.BlockSpec((1, _D), full),
            pl.BlockSpec((1, _D), full),
        ],
        out_specs=pl.BlockSpec((_BLK, _D), lambda i: (i, 0)),
        out_shape=jax.ShapeDtypeStruct((_N, _D), jnp.float32),
    )(h, a0, a1, W1, b1, W2, b2, scale, gamma, beta)


def kernel(h, edge_index, W1, b1, W2, b2, eps, gamma, beta):
    src = edge_index[0]
    dst = edge_index[1]
    idx2 = jnp.concatenate([src * 2, src * 2 + 1])
    h2 = h.reshape(2 * _N, _HALF)
    zeros = jnp.zeros((_WB_ROWS, _HALF), jnp.float32)
    agg2 = _sc_agg(h2, idx2, dst, zeros)
    scale = jnp.reshape(1.0 + eps, (1, 1))
    return _dense(h, agg2[0], agg2[1],
                  W1, b1.reshape(1, _D), W2, b2.reshape(1, _D),
                  scale, gamma.reshape(1, _D), beta.reshape(1, _D))


# speedup vs baseline: 7.6106x; 1.4579x over previous
import functools

import jax
import jax.numpy as jnp
from jax import lax
from jax.experimental import pallas as pl
from jax.experimental.pallas import tpu as pltpu
from jax.experimental.pallas import tpu_sc as plsc

_N = 10000
_D = 256
_E = 160000
_HALF = _D // 2
_NTILES = 16
_EDGES_PER_TILE = _E // _NTILES
_CHUNK = 80
_NCHUNK = _EDGES_PER_TILE // _CHUNK
_WB_TILES = 10
_WB_ROWS = _N // _WB_TILES


def _make_sc_agg():
    mesh = plsc.VectorSubcoreMesh(core_axis_name="c", subcore_axis_name="s")

    @functools.partial(
        pl.kernel,
        mesh=mesh,
        out_type=jax.ShapeDtypeStruct((2, _N, _HALF), jnp.float32),
        scratch_types=[
            pltpu.VMEM((_EDGES_PER_TILE,), jnp.int32),
            pltpu.VMEM((_EDGES_PER_TILE,), jnp.int32),
            pltpu.VMEM((_CHUNK, _HALF), jnp.float32),
            pltpu.VMEM((_CHUNK, _HALF), jnp.float32),
            pltpu.VMEM_SHARED((_N, _HALF), jnp.float32),
            pltpu.SemaphoreType.DMA,
            pltpu.SemaphoreType.DMA,
            pltpu.SemaphoreType.DMA,
            pltpu.SemaphoreType.DMA,
        ],
    )
    def sc_agg(h2_hbm, idx2_hbm, dst_hbm, zeros_hbm, out_hbm,
               idx_big, dst_big, rows0, rows1, acc_sh, g0, g1, s0, s1):
        c = lax.axis_index("c")
        s = lax.axis_index("s")

        pltpu.sync_copy(
            idx2_hbm.at[pl.ds(c * _E + s * _EDGES_PER_TILE, _EDGES_PER_TILE)],
            idx_big)
        pltpu.sync_copy(dst_hbm.at[pl.ds(s * _EDGES_PER_TILE, _EDGES_PER_TILE)],
                        dst_big)

        def islice(ref, k):
            return ref.at[pl.ds(k * _CHUNK, _CHUNK)]

        pltpu.async_copy(h2_hbm.at[islice(idx_big, 0)], rows0, g0)

        row0 = s * _WB_ROWS

        @pl.when(s < _WB_TILES)
        def _zero():
            pltpu.sync_copy(zeros_hbm, acc_sh.at[pl.ds(row0, _WB_ROWS)])

        plsc.subcore_barrier()

        def gather(k, rows, sem):
            pltpu.async_copy(h2_hbm.at[islice(idx_big, k)], rows, sem)

        def drain_gather(k, rows, sem):
            pltpu.make_async_copy(h2_hbm.at[islice(idx_big, k)], rows, sem).wait()

        def scatter(k, rows, sem):
            pltpu.async_copy(rows, acc_sh.at[islice(dst_big, k)], sem, add=True)

        def drain_scatter(k, rows, sem):
            pltpu.make_async_copy(rows, acc_sh.at[islice(dst_big, k)], sem).wait()

        def pair_body(j, carry):
            k0 = 2 * j
            k1 = k0 + 1

            @pl.when(j > 0)
            def _():
                drain_scatter(k1 - 2, rows1, s1)

            gather(k1, rows1, g1)
            drain_gather(k0, rows0, g0)
            scatter(k0, rows0, s0)
            drain_scatter(k0, rows0, s0)
            gather(k0 + 2, rows0, g0)
            drain_gather(k1, rows1, g1)
            scatter(k1, rows1, s1)
            return carry

        lax.fori_loop(0, (_NCHUNK - 1) // 2, pair_body, 0)
        drain_scatter(_NCHUNK - 2, rows1, s1)
        drain_gather(_NCHUNK - 1, rows0, g0)
        scatter(_NCHUNK - 1, rows0, s0)
        drain_scatter(_NCHUNK - 1, rows0, s0)
        plsc.subcore_barrier()

        @pl.when(s < _WB_TILES)
        def _writeback():
            pltpu.sync_copy(acc_sh.at[pl.ds(row0, _WB_ROWS)],
                            out_hbm.at[c, pl.ds(row0, _WB_ROWS)])

    return sc_agg


_sc_agg = _make_sc_agg()


def _dense_body(h_ref, a0_ref, a1_ref, w1_ref, b1_ref, w2_ref, b2_ref,
                sc_ref, gamma_ref, beta_ref, out_ref):
    h = h_ref[...]
    agg = jnp.concatenate([a0_ref[...], a1_ref[...]], axis=1)
    z = h * sc_ref[...] + agg
    t = jnp.maximum(jnp.dot(z, w1_ref[...], preferred_element_type=jnp.float32)
                    + b1_ref[...], 0.0)
    r = jnp.dot(t, w2_ref[...], preferred_element_type=jnp.float32) \
        + b2_ref[...] + h
    mu = jnp.mean(r, axis=1, keepdims=True)
    d = r - mu
    var = jnp.mean(d * d, axis=1, keepdims=True)
    ln = d * lax.rsqrt(var + 1e-5) * gamma_ref[...] + beta_ref[...]
    out_ref[...] = jnp.maximum(ln, 0.0)


_BLK = 1000


def _dense(h, a0, a1, W1, b1, W2, b2, scale, gamma, beta):
    nblk = _N // _BLK
    full = lambda i: (0, 0)
    return pl.pallas_call(
        _dense_body,
        grid=(nblk,),
        in_specs=[
            pl.BlockSpec((_BLK, _D), lambda i: (i, 0)),
            pl.BlockSpec((_BLK, _HALF), lambda i: (i, 0)),
            pl.BlockSpec((_BLK, _HALF), lambda i: (i, 0)),
            pl.BlockSpec((_D, _D), full),
            pl.BlockSpec((1, _D), full),
            pl.BlockSpec((_D, _D), full),
            pl.BlockSpec((1, _D), full),
            pl.BlockSpec((1, 1), full),
            pl.BlockSpec((1, _D), full),
            pl.BlockSpec((1, _D), full),
        ],
        out_specs=pl.BlockSpec((_BLK, _D), lambda i: (i, 0)),
        out_shape=jax.ShapeDtypeStruct((_N, _D), jnp.float32),
    )(h, a0, a1, W1, b1, W2, b2, scale, gamma, beta)


def kernel(h, edge_index, W1, b1, W2, b2, eps, gamma, beta):
    src = edge_index[0]
    dst = edge_index[1]
    idx2 = jnp.concatenate([src * 2, src * 2 + 1])
    h2 = h.reshape(2 * _N, _HALF)
    zeros = jnp.zeros((_WB_ROWS, _HALF), jnp.float32)
    agg2 = _sc_agg(h2, idx2, dst, zeros)
    scale = jnp.reshape(1.0 + eps, (1, 1))
    return _dense(h, agg2[0], agg2[1],
                  W1, b1.reshape(1, _D), W2, b2.reshape(1, _D),
                  scale, gamma.reshape(1, _D), beta.reshape(1, _D))


# speedup vs baseline: 8.8377x; 1.1612x over previous
import functools

import jax
import jax.numpy as jnp
from jax import lax
from jax.experimental import pallas as pl
from jax.experimental.pallas import tpu as pltpu
from jax.experimental.pallas import tpu_sc as plsc

_N = 10000
_D = 256
_E = 160000
_HALF = _D // 2
_NTILES = 16
_EDGES_PER_TILE = _E // _NTILES
_CHUNK = 80
_NCHUNK = _EDGES_PER_TILE // _CHUNK
_WB_TILES = 10
_WB_ROWS = _N // _WB_TILES


def _make_sc_agg():
    mesh = plsc.VectorSubcoreMesh(core_axis_name="c", subcore_axis_name="s")

    @functools.partial(
        pl.kernel,
        mesh=mesh,
        out_type=jax.ShapeDtypeStruct((2, _N, _HALF), jnp.float32),
        scratch_types=[
            pltpu.VMEM((_EDGES_PER_TILE,), jnp.int32),
            pltpu.VMEM((_EDGES_PER_TILE,), jnp.int32),
            pltpu.VMEM((_CHUNK, _HALF), jnp.float32),
            pltpu.VMEM((_CHUNK, _HALF), jnp.float32),
            pltpu.VMEM((_CHUNK, _HALF), jnp.float32),
            pltpu.VMEM_SHARED((_N, _HALF), jnp.float32),
            pltpu.SemaphoreType.DMA,
            pltpu.SemaphoreType.DMA,
            pltpu.SemaphoreType.DMA,
            pltpu.SemaphoreType.DMA,
            pltpu.SemaphoreType.DMA,
            pltpu.SemaphoreType.DMA,
        ],
    )
    def sc_agg(h2_hbm, idx2_hbm, dst_hbm, zeros_hbm, out_hbm,
               idx_big, dst_big, rows0, rows1, rows2, acc_sh,
               g0, g1, g2, s0, s1, s2):
        c = lax.axis_index("c")
        s = lax.axis_index("s")

        pltpu.sync_copy(
            idx2_hbm.at[pl.ds(c * _E + s * _EDGES_PER_TILE, _EDGES_PER_TILE)],
            idx_big)
        pltpu.sync_copy(dst_hbm.at[pl.ds(s * _EDGES_PER_TILE, _EDGES_PER_TILE)],
                        dst_big)

        def islice(ref, k):
            return ref.at[pl.ds(k * _CHUNK, _CHUNK)]

        pltpu.async_copy(h2_hbm.at[islice(idx_big, 0)], rows0, g0)

        row0 = s * _WB_ROWS

        @pl.when(s < _WB_TILES)
        def _zero():
            pltpu.sync_copy(zeros_hbm, acc_sh.at[pl.ds(row0, _WB_ROWS)])

        plsc.subcore_barrier()

        def gather(k, rows, sem):
            pltpu.async_copy(h2_hbm.at[islice(idx_big, k)], rows, sem)

        def drain_gather(k, rows, sem):
            pltpu.make_async_copy(h2_hbm.at[islice(idx_big, k)], rows, sem).wait()

        def scatter(k, rows, sem):
            pltpu.async_copy(rows, acc_sh.at[islice(dst_big, k)], sem, add=True)

        def drain_scatter(k, rows, sem):
            pltpu.make_async_copy(rows, acc_sh.at[islice(dst_big, k)], sem).wait()

        rows = (rows0, rows1, rows2)
        gsem = (g0, g1, g2)
        ssem = (s0, s1, s2)

        def triple_body(j, carry):
            for r in range(3):
                k = 3 * j + r
                cur = r
                prev = (r + 2) % 3

                @pl.when(k >= 3)
                def _(k=k, cur=cur):
                    drain_scatter(k - 3, rows[cur], ssem[cur])

                @pl.when(jnp.logical_and(k >= 1, k < _NCHUNK))
                def _(k=k, cur=cur):
                    gather(k, rows[cur], gsem[cur])

                @pl.when(k >= 1)
                def _(k=k, prev=prev):
                    drain_gather(k - 1, rows[prev], gsem[prev])
                    scatter(k - 1, rows[prev], ssem[prev])

            return carry

        lax.fori_loop(0, (_NCHUNK + 1) // 3, triple_body, 0)
        drain_scatter(_NCHUNK - 2, rows[(_NCHUNK - 2) % 3], ssem[(_NCHUNK - 2) % 3])
        drain_scatter(_NCHUNK - 1, rows[(_NCHUNK - 1) % 3], ssem[(_NCHUNK - 1) % 3])
        plsc.subcore_barrier()

        @pl.when(s < _WB_TILES)
        def _writeback():
            pltpu.sync_copy(acc_sh.at[pl.ds(row0, _WB_ROWS)],
                            out_hbm.at[c, pl.ds(row0, _WB_ROWS)])

    return sc_agg


_sc_agg = _make_sc_agg()


def _dense_body(h_ref, a0_ref, a1_ref, w1_ref, b1_ref, w2_ref, b2_ref,
                sc_ref, gamma_ref, beta_ref, out_ref):
    h = h_ref[...]
    agg = jnp.concatenate([a0_ref[0], a1_ref[0]], axis=1)
    z = h * sc_ref[...] + agg
    t = jnp.maximum(jnp.dot(z, w1_ref[...], preferred_element_type=jnp.float32)
                    + b1_ref[...], 0.0)
    r = jnp.dot(t, w2_ref[...], preferred_element_type=jnp.float32) \
        + b2_ref[...] + h
    mu = jnp.mean(r, axis=1, keepdims=True)
    d = r - mu
    var = jnp.mean(d * d, axis=1, keepdims=True)
    ln = d * lax.rsqrt(var + 1e-5) * gamma_ref[...] + beta_ref[...]
    out_ref[...] = jnp.maximum(ln, 0.0)


_BLK = 1000


def _dense(h, agg2, W1, b1, W2, b2, scale, gamma, beta):
    nblk = _N // _BLK
    full = lambda i: (0, 0)
    return pl.pallas_call(
        _dense_body,
        grid=(nblk,),
        in_specs=[
            pl.BlockSpec((_BLK, _D), lambda i: (i, 0)),
            pl.BlockSpec((1, _BLK, _HALF), lambda i: (0, i, 0)),
            pl.BlockSpec((1, _BLK, _HALF), lambda i: (1, i, 0)),
            pl.BlockSpec((_D, _D), full),
            pl.BlockSpec((1, _D), full),
            pl.BlockSpec((_D, _D), full),
            pl.BlockSpec((1, _D), full),
            pl.BlockSpec((1, 1), full),
            pl.BlockSpec((1, _D), full),
            pl.BlockSpec((1, _D), full),
        ],
        out_specs=pl.BlockSpec((_BLK, _D), lambda i: (i, 0)),
        out_shape=jax.ShapeDtypeStruct((_N, _D), jnp.float32),
    )(h, agg2, agg2, W1, b1, W2, b2, scale, gamma, beta)


def kernel(h, edge_index, W1, b1, W2, b2, eps, gamma, beta):
    src = edge_index[0]
    dst = edge_index[1]
    idx2 = jnp.concatenate([src * 2, src * 2 + 1])
    h2 = h.reshape(2 * _N, _HALF)
    zeros = jnp.zeros((_WB_ROWS, _HALF), jnp.float32)
    agg2 = _sc_agg(h2, idx2, dst, zeros)
    scale = jnp.reshape(1.0 + eps, (1, 1))
    return _dense(h, agg2,
                  W1, b1.reshape(1, _D), W2, b2.reshape(1, _D),
                  scale, gamma.reshape(1, _D), beta.reshape(1, _D))
